# Initial kernel scaffold; baseline (speedup 1.0000x reference)
import jax
import jax.numpy as jnp
from jax.experimental import pallas as pl


def kernel(x_ue, x_ap, ei_up, ea_up, ei_dn, ea_dn, params):
    raise NotImplementedError("write your pallas kernel here")


# speedup vs baseline: 2.0176x; 2.0176x over previous
import jax
import jax.numpy as jnp
from jax import lax
from jax.experimental import pallas as pl
from jax.experimental.pallas import tpu as pltpu
from jax.experimental.pallas import tpu_sc as plsc

N = 50000
E = 800000
NC = 2
NS = 16
NW = NC * NS
EPT = 25600
E_PAD = EPT * NW
G_CH = 3200
G_NCH = EPT // G_CH
S_CH = 1024
S_NCH = EPT // S_CH
N_PAD = 50176
STRIPE = N_PAD // NS
HSTRIPE = STRIPE // 2
D = 32
BE = 8192
BN = 5000

_mesh = lambda: plsc.VectorSubcoreMesh(core_axis_name="c", subcore_axis_name="s")


def _make_gather(num_pairs):
    out_type = [jax.ShapeDtypeStruct((E_PAD * 2,), jnp.float32)
                for _ in range(num_pairs)]
    scratch = [
        pltpu.VMEM((2 * N,), jnp.float32),
        pltpu.VMEM((G_CH,), jnp.int32),
        pltpu.VMEM((2 * G_CH,), jnp.float32)
    ]

    def body(*refs):
        tabs = refs[0:2 * num_pairs:2]
        idxs = refs[1:2 * num_pairs:2]
        outs = refs[2 * num_pairs:3 * num_pairs]
        tab_v, idx_v, out_v = refs[3 * num_pairs:]
        wid = lax.axis_index("s") * NC + lax.axis_index("c")
        lane = lax.iota(jnp.int32, 16)
        for p in range(num_pairs):
            pltpu.sync_copy(tabs[p], tab_v)
            for ch in range(G_NCH):
                base = wid * EPT + ch * G_CH
                pltpu.sync_copy(idxs[p].at[pl.ds(base, G_CH)], idx_v)

                def grp(g, carry):
                    i16 = idx_v[pl.ds(g * 16, 16)]
                    a0 = i16 * 2
                    v0 = plsc.load_gather(tab_v, [a0])
                    v1 = plsc.load_gather(tab_v, [a0 + 1])
                    pos = g * 32 + lane * 2
                    plsc.store_scatter(out_v, [pos], v0)
                    plsc.store_scatter(out_v, [pos + 1], v1)
                    return carry

                lax.fori_loop(0, G_CH // 16, grp, None)
                pltpu.sync_copy(out_v, outs[p].at[pl.ds(2 * base, 2 * G_CH)])

    return pl.kernel(
        body, out_type=out_type, mesh=_mesh(), scratch_types=scratch,
        compiler_params=pltpu.CompilerParams(needs_layout_passes=False))


def _gather(pairs):
    fn = _make_gather(len(pairs))
    args = []
    for tab, idx in pairs:
        args.append(tab.reshape(-1))
        args.append(idx)
    outs = fn(*args)
    return [o.reshape(E_PAD, 2) for o in outs]


def _make_scatter(with_counts):
    ept = E_PAD // NS
    snch = ept // S_CH
    ZB = STRIPE // 8
    out_type = [jax.ShapeDtypeStruct((N_PAD, D), jnp.float32)]
    scratch = [
        pltpu.VMEM_SHARED((N_PAD, D), jnp.float32),
        pltpu.VMEM((ZB, D), jnp.float32),
        pltpu.VMEM((S_CH // 4, D), jnp.float32),
        pltpu.VMEM((8, 128), jnp.int32),
    ]
    if with_counts:
        out_type.append(jax.ShapeDtypeStruct((N_PAD,), jnp.float32))
        scratch += [
            pltpu.VMEM_SHARED((N_PAD,), jnp.float32),
            pltpu.VMEM((STRIPE,), jnp.float32),
            pltpu.VMEM((128,), jnp.float32),
        ]

    def body(*refs):
        if with_counts:
            (msg, dst2d, z2d, z1d, ones_in, aggr_out, cnt_out,
             sp_a, zb, msg_v, idx_v, sp_c, cnt_b, ones_v) = refs
        else:
            (msg, dst2d, z2d, aggr_out, sp_a, zb, msg_v, idx_v) = refs
        s = lax.axis_index("s")
        pltpu.sync_copy(z2d, zb)
        for k in range(8):
            pltpu.sync_copy(zb, sp_a.at[pl.ds(s * STRIPE + k * ZB, ZB)])
        if with_counts:
            pltpu.sync_copy(z1d, cnt_b)
            pltpu.sync_copy(cnt_b, sp_c.at[pl.ds(s * STRIPE, STRIPE)])
            pltpu.sync_copy(ones_in, ones_v)
        plsc.subcore_barrier()

        def chunk(t, carry):
            ebase = s * ept + t * S_CH
            rbase = s * (ept // 128) + t * 8
            pltpu.sync_copy(dst2d.at[pl.ds(rbase, 8)], idx_v)
            for q in range(4):
                pltpu.sync_copy(msg.at[pl.ds(ebase + q * 256, 256)], msg_v)
                for j in range(2):
                    pltpu.sync_copy(msg_v.at[pl.ds(j * 128, 128)],
                                    sp_a.at[idx_v.at[q * 2 + j]], add=True)
                    if with_counts:
                        pltpu.sync_copy(ones_v,
                                        sp_c.at[idx_v.at[q * 2 + j]],
                                        add=True)
            return carry

        lax.fori_loop(0, snch, chunk, None)
        plsc.subcore_barrier()
        for k in range(8):
            pltpu.sync_copy(sp_a.at[pl.ds(s * STRIPE + k * ZB, ZB)], zb)
            pltpu.sync_copy(zb, aggr_out.at[pl.ds(s * STRIPE + k * ZB, ZB)])
        if with_counts:
            pltpu.sync_copy(sp_c.at[pl.ds(s * STRIPE, STRIPE)], cnt_b)
            pltpu.sync_copy(cnt_b, cnt_out.at[pl.ds(s * STRIPE, STRIPE)])

    mesh = plsc.VectorSubcoreMesh(core_axis_name="c", subcore_axis_name="s",
                                  num_cores=1)
    return pl.kernel(
        body, out_type=out_type, mesh=mesh, scratch_types=scratch,
        compiler_params=pltpu.CompilerParams(use_tc_tiling_on_sc=False))


def _scatter_counts(msg, dst2d):
    z2d = jnp.zeros((STRIPE // 8, D), jnp.float32)
    z1d = jnp.zeros((STRIPE,), jnp.float32)
    ones = jnp.ones((128,), jnp.float32)
    aggr, cnt = _make_scatter(True)(msg, dst2d, z2d, z1d, ones)
    return (aggr, cnt.reshape(N_PAD, 1))


def _scatter(msg, dst2d):
    z2d = jnp.zeros((STRIPE // 8, D), jnp.float32)
    (aggr,) = _make_scatter(False)(msg, dst2d, z2d)
    return aggr


def _eblk():
    return pl.BlockSpec((BE, 2), lambda i: (i, 0))


def _wfull(w):
    return pl.BlockSpec(w.shape, lambda i: (0,) * w.ndim)


def _edge_up_tc(src_f, dst_f, ea, p):
    Wam1 = p["am1"]["W"]
    Ws, Wd, We = Wam1[0:2], Wam1[2:4], Wam1[4:6]
    b1 = p["am1"]["b"].reshape(1, -1)
    W2 = p["am2"]["W"]
    b2 = p["am2"]["b"].reshape(1, -1)
    Wue = p["ln_ue"]["W"]
    bue = p["ln_ue"]["b"].reshape(1, -1)
    Wle = p["le_up"]["W"]
    Wl0, Wl1 = Wle[0:1], Wle[1:2]
    bl = p["le_up"]["b"].reshape(1, -1)

    def body(s_ref, d_ref, a_ref, ws, wd, we, b1r, w2, b2r, wue, buer,
             wl0, wl1, blr, msg_ref, ean_ref):
        s = s_ref[...]
        d = d_ref[...]
        a = a_ref[...]
        h = jnp.maximum(
            s @ ws[...] + d @ wd[...] + a @ we[...] + b1r[...], 0.0)
        eo = jax.nn.sigmoid(h @ w2[...] + b2r[...])
        a0 = a[:, 0:1]
        ean_ref[...] = jnp.concatenate([a0, eo], axis=1)
        msg_ref[...] = (
            jnp.maximum(s @ wue[...] + buer[...], 0.0)
            + jnp.maximum(a0 @ wl0[...] + eo @ wl1[...] + blr[...], 0.0))

    ws = (Ws, Wd, We, b1, W2, b2, Wue, bue, Wl0, Wl1, bl)
    return pl.pallas_call(
        body,
        grid=(E_PAD // BE,),
        in_specs=[_eblk(), _eblk(), _eblk()] + [_wfull(w) for w in ws],
        out_specs=[pl.BlockSpec((BE, D), lambda i: (i, 0)),
                   pl.BlockSpec((BE, 2), lambda i: (i, 0))],
        out_shape=[jax.ShapeDtypeStruct((E_PAD, D), jnp.float32),
                   jax.ShapeDtypeStruct((E_PAD, 2), jnp.float32)],
    )(src_f, dst_f, ea, *ws)


def _edge_dn_first_tc(ea, p):
    Wld = p["le_dn"]["W"]
    bld = p["le_dn"]["b"].reshape(1, -1)

    def body(a_ref, w, br, msg_ref):
        msg_ref[...] = jnp.maximum(a_ref[...] @ w[...] + br[...], 0.0)

    return pl.pallas_call(
        body,
        grid=(E_PAD // BE,),
        in_specs=[_eblk(), _wfull(Wld), _wfull(bld)],
        out_specs=pl.BlockSpec((BE, D), lambda i: (i, 0)),
        out_shape=jax.ShapeDtypeStruct((E_PAD, D), jnp.float32),
    )(ea, Wld, bld)


def _edge_dn_later_tc(src_f, ea, p):
    Wap = p["ln_ap"]["W"]
    bap = p["ln_ap"]["b"].reshape(1, -1)
    Wld = p["le_dn"]["W"]
    bld = p["le_dn"]["b"].reshape(1, -1)

    def body(s_ref, a_ref, w1, b1r, w2, b2r, msg_ref):
        msg_ref[...] = (
            jnp.maximum(s_ref[...] @ w1[...] + b1r[...], 0.0)
            + jnp.maximum(a_ref[...] @ w2[...] + b2r[...], 0.0))

    return pl.pallas_call(
        body,
        grid=(E_PAD // BE,),
        in_specs=[_eblk(), _eblk(), _wfull(Wap), _wfull(bap),
                  _wfull(Wld), _wfull(bld)],
        out_specs=pl.BlockSpec((BE, D), lambda i: (i, 0)),
        out_shape=jax.ShapeDtypeStruct((E_PAD, D), jnp.float32),
    )(src_f, ea, Wap, bap, Wld, bld)


def _node_tc(x, aggr, cnt, pn, p1, p2):
    Wn = pn["W"]
    bn = pn["b"].reshape(1, -1)
    W1 = p1["W"]
    W1x, W1a = W1[0:2], W1[2:34]
    b1 = p1["b"].reshape(1, -1)
    W2 = p2["W"]
    b2 = p2["b"].reshape(1, -1)

    def body(x_ref, a_ref, c_ref, wn, bnr, w1x, w1a, b1r, w2, b2r, o_ref):
        x = x_ref[...]
        c = jnp.maximum(c_ref[...], 1.0)
        t = a_ref[...] / c + jnp.maximum(x @ wn[...] + bnr[...], 0.0)
        h = jnp.maximum(x @ w1x[...] + t @ w1a[...] + b1r[...], 0.0)
        pw = jax.nn.sigmoid(h @ w2[...] + b2r[...])
        o_ref[...] = jnp.concatenate([x[:, 0:1], pw], axis=1)

    ws = (Wn, bn, W1x, W1a, b1, W2, b2)
    nb = lambda w: pl.BlockSpec((BN, w), lambda i: (i, 0))
    return pl.pallas_call(
        body,
        grid=(N // BN,),
        in_specs=[nb(2), nb(D), nb(1)] + [_wfull(w) for w in ws],
        out_specs=nb(2),
        out_shape=jax.ShapeDtypeStruct((N, 2), jnp.float32),
    )(x, aggr, cnt, *ws)


def kernel(x_ue, x_ap, ei_up, ea_up, ei_dn, ea_dn, params):
    pad = E_PAD - E
    i32 = jnp.int32
    src_up = jnp.concatenate([ei_up[0].astype(i32), jnp.zeros((pad,), i32)])
    dst_up = jnp.concatenate([ei_up[1].astype(i32), jnp.full((pad,), N, i32)])
    src_dn = jnp.concatenate([ei_dn[0].astype(i32), jnp.zeros((pad,), i32)])
    dst_dn = jnp.concatenate([ei_dn[1].astype(i32), jnp.full((pad,), N, i32)])
    dst2d_up = dst_up.reshape(E_PAD // 128, 128)
    dst2d_dn = dst_dn.reshape(E_PAD // 128, 128)
    ea_up_p = jnp.concatenate([ea_up, jnp.zeros((pad, 2), jnp.float32)])
    ea_dn_p = jnp.concatenate([ea_dn, jnp.zeros((pad, 2), jnp.float32)])

    p0, p1 = params[0], params[1]

    gs, gd = _gather([(x_ue, src_up), (x_ap, dst_up)])
    msg_up, ean_p = _edge_up_tc(gs, gd, ea_up_p, p0)
    aggr_up, cnt_up = _scatter_counts(msg_up, dst2d_up)
    x_ap1 = _node_tc(x_ap, aggr_up, cnt_up, p0["ln_ap"], p0["pm1"], p0["pm2"])
    msg_dn = _edge_dn_first_tc(ea_dn_p, p0)
    aggr_dn, cnt_dn = _scatter_counts(msg_dn, dst2d_dn)
    x_ue1 = _node_tc(x_ue, aggr_dn, cnt_dn, p0["ln_ue"], p0["pm1"], p0["pm2"])

    gs2, gd2, gdn2 = _gather([(x_ue1, src_up), (x_ap1, dst_up),
                              (x_ap1, src_dn)])
    msg_up2, ean2_p = _edge_up_tc(gs2, gd2, ean_p, p1)
    aggr_up2 = _scatter(msg_up2, dst2d_up)
    x_ap2 = _node_tc(x_ap1, aggr_up2, cnt_up,
                     p1["ln_ap"], p1["pm1"], p1["pm2"])
    msg_dn2 = _edge_dn_later_tc(gdn2, ea_dn_p, p1)
    aggr_dn2 = _scatter(msg_dn2, dst2d_dn)
    x_ue2 = _node_tc(x_ue1, aggr_dn2, cnt_dn,
                     p1["ln_ue"], p1["pm1"], p1["pm2"])

    return (x_ue2, x_ap2, ean2_p[:E], ea_dn)


# speedup vs baseline: 3.5346x; 1.7519x over previous
import jax
import jax.numpy as jnp
from jax import lax
from jax.experimental import pallas as pl
from jax.experimental.pallas import tpu as pltpu
from jax.experimental.pallas import tpu_sc as plsc

N = 50000
E = 800000
NC = 2
NS = 16
NW = NC * NS
EPT = 25600
E_PAD = EPT * NW
G_CH = 3200
G_NCH = EPT // G_CH
S_CH = 1024
N_PAD = 50176
STRIPE = N_PAD // NS
D = 32
EB = E_PAD // 64
BB = 640
BN = 5000


def _make_gather(groups):
    n_out = sum(groups)
    out_type = [jax.ShapeDtypeStruct((E_PAD * 2,), jnp.float32)
                for _ in range(n_out)]
    scratch = [
        pltpu.VMEM((N_PAD,), jnp.float32),
        pltpu.VMEM((N_PAD,), jnp.float32),
        pltpu.VMEM((G_CH,), jnp.int32),
        pltpu.VMEM((2 * G_CH,), jnp.float32)
    ]

    def body(*refs):
        n_in = 2 * len(groups) + n_out
        ins = refs[:n_in]
        outs = refs[n_in:n_in + n_out]
        t0_v, t1_v, idx_v, out_v = refs[n_in + n_out:]
        wid = lax.axis_index("s") * NC + lax.axis_index("c")
        lane = lax.iota(jnp.int32, 16)
        pos_in = 0
        oi = 0
        for g, n_idx in enumerate(groups):
            tab0, tab1 = ins[pos_in], ins[pos_in + 1]
            idxs = ins[pos_in + 2:pos_in + 2 + n_idx]
            pos_in += 2 + n_idx
            pltpu.sync_copy(tab0, t0_v)
            pltpu.sync_copy(tab1, t1_v)
            for idx in idxs:
                out = outs[oi]
                oi += 1
                for ch in range(G_NCH):
                    base = wid * EPT + ch * G_CH
                    pltpu.sync_copy(idx.at[pl.ds(base, G_CH)], idx_v)

                    def grp(g2, carry):
                        i16 = idx_v[pl.ds(g2 * 16, 16)]
                        v0 = plsc.load_gather(t0_v, [i16])
                        v1 = plsc.load_gather(t1_v, [i16])
                        pos = g2 * 32 + lane * 2
                        plsc.store_scatter(out_v, [pos], v0)
                        plsc.store_scatter(out_v, [pos + 1], v1)
                        return carry

                    lax.fori_loop(0, G_CH // 16, grp, None)
                    pltpu.sync_copy(out_v, out.at[pl.ds(2 * base, 2 * G_CH)])

    return pl.kernel(
        body,
        out_type=out_type,
        mesh=plsc.VectorSubcoreMesh(core_axis_name="c", subcore_axis_name="s"),
        scratch_types=scratch,
        compiler_params=pltpu.CompilerParams(needs_layout_passes=False))


def _gather(groups_args, groups):
    outs = _make_gather(groups)(*groups_args)
    return [o.reshape(EB, 128) for o in outs]


def _make_scatter(with_counts):
    ept = E_PAD // NS
    snch = ept // S_CH
    ZB = STRIPE // 8
    out_type = [jax.ShapeDtypeStruct((N_PAD, D), jnp.float32)]
    scratch = [
        pltpu.VMEM_SHARED((N_PAD, D), jnp.float32),
        pltpu.VMEM((ZB, D), jnp.float32),
        pltpu.VMEM((S_CH // 4, D), jnp.float32),
        pltpu.VMEM((8, 128), jnp.int32),
    ]
    if with_counts:
        out_type.append(jax.ShapeDtypeStruct((N_PAD,), jnp.float32))
        scratch += [
            pltpu.VMEM_SHARED((N_PAD,), jnp.float32),
            pltpu.VMEM((STRIPE,), jnp.float32),
            pltpu.VMEM((128,), jnp.float32),
        ]

    def body(*refs):
        if with_counts:
            (msg, dst2d, z2d, z1d, ones_in, aggr_out, cnt_out,
             sp_a, zb, msg_v, idx_v, sp_c, cnt_b, ones_v) = refs
        else:
            (msg, dst2d, z2d, aggr_out, sp_a, zb, msg_v, idx_v) = refs
        s = lax.axis_index("s")
        pltpu.sync_copy(z2d, zb)
        for k in range(8):
            pltpu.sync_copy(zb, sp_a.at[pl.ds(s * STRIPE + k * ZB, ZB)])
        if with_counts:
            pltpu.sync_copy(z1d, cnt_b)
            pltpu.sync_copy(cnt_b, sp_c.at[pl.ds(s * STRIPE, STRIPE)])
            pltpu.sync_copy(ones_in, ones_v)
        plsc.subcore_barrier()

        def chunk(t, carry):
            ebase = s * ept + t * S_CH
            rbase = s * (ept // 128) + t * 8
            pltpu.sync_copy(dst2d.at[pl.ds(rbase, 8)], idx_v)
            for q in range(4):
                pltpu.sync_copy(msg.at[pl.ds(ebase + q * 256, 256)], msg_v)
                for j in range(2):
                    pltpu.sync_copy(msg_v.at[pl.ds(j * 128, 128)],
                                    sp_a.at[idx_v.at[q * 2 + j]], add=True)
                    if with_counts:
                        pltpu.sync_copy(ones_v,
                                        sp_c.at[idx_v.at[q * 2 + j]],
                                        add=True)
            return carry

        lax.fori_loop(0, snch, chunk, None)
        plsc.subcore_barrier()
        for k in range(8):
            pltpu.sync_copy(sp_a.at[pl.ds(s * STRIPE + k * ZB, ZB)], zb)
            pltpu.sync_copy(zb, aggr_out.at[pl.ds(s * STRIPE + k * ZB, ZB)])
        if with_counts:
            pltpu.sync_copy(sp_c.at[pl.ds(s * STRIPE, STRIPE)], cnt_b)
            pltpu.sync_copy(cnt_b, cnt_out.at[pl.ds(s * STRIPE, STRIPE)])

    mesh = plsc.VectorSubcoreMesh(core_axis_name="c", subcore_axis_name="s",
                                  num_cores=1)
    return pl.kernel(
        body, out_type=out_type, mesh=mesh, scratch_types=scratch,
        compiler_params=pltpu.CompilerParams(use_tc_tiling_on_sc=False))


def _scatter_counts(msg_packed, dst2d):
    msg = msg_packed.reshape(E_PAD, D)
    z2d = jnp.zeros((STRIPE // 8, D), jnp.float32)
    z1d = jnp.zeros((STRIPE,), jnp.float32)
    ones = jnp.ones((128,), jnp.float32)
    aggr, cnt = _make_scatter(True)(msg, dst2d, z2d, z1d, ones)
    return (aggr, cnt.reshape(N_PAD, 1))


def _scatter(msg_packed, dst2d):
    msg = msg_packed.reshape(E_PAD, D)
    z2d = jnp.zeros((STRIPE // 8, D), jnp.float32)
    (aggr,) = _make_scatter(False)(msg, dst2d, z2d)
    return aggr


def _bd(w, g=64):
    return jnp.kron(jnp.eye(g, dtype=jnp.float32), w)


def _bt(b, g=64):
    return jnp.tile(b, (g,)).reshape(1, -1)


def _wfull(w):
    return pl.BlockSpec(w.shape, lambda i: (0,) * w.ndim)


def _pblk(width):
    return pl.BlockSpec((BB, width), lambda i: (i, 0))


def _edge_up_tc(g_s, g_d, a_int, p):
    Wam1 = p["am1"]["W"]
    A = _bd(Wam1[0:2])
    B = _bd(Wam1[2:4])
    C = _bd(Wam1[4:6])
    b1 = _bt(p["am1"]["b"])
    W2 = _bd(p["am2"]["W"])
    b2 = _bt(p["am2"]["b"])
    Wue = _bd(p["ln_ue"]["W"])
    bue = _bt(p["ln_ue"]["b"])
    Wle = p["le_up"]["W"]
    Wl0 = _bd(jnp.concatenate([Wle[0:1], jnp.zeros_like(Wle[0:1])]))
    Wl1 = _bd(Wle[1:2])
    bl = _bt(p["le_up"]["b"])
    eye = jnp.eye(64, dtype=jnp.float32)
    eodd = jnp.kron(eye, jnp.array([[0.0, 1.0]], jnp.float32))
    meven = jnp.tile(jnp.array([1.0, 0.0], jnp.float32), (64,)).reshape(1, -1)

    def body(s_ref, d_ref, a_ref, wa, wb, wc, b1r, w2, b2r, wue, buer,
             wl0, wl1, blr, eo_r, me_r, msg_ref, a2_ref):
        s = s_ref[...]
        d = d_ref[...]
        a = a_ref[...]
        h = jnp.maximum(
            s @ wa[...] + d @ wb[...] + a @ wc[...] + b1r[...], 0.0)
        eo = jax.nn.sigmoid(h @ w2[...] + b2r[...])
        msg_ref[...] = (
            jnp.maximum(s @ wue[...] + buer[...], 0.0)
            + jnp.maximum(a @ wl0[...] + eo @ wl1[...] + blr[...], 0.0))
        a2_ref[...] = a * me_r[...] + eo @ eo_r[...]

    ws = (A, B, C, b1, W2, b2, Wue, bue, Wl0, Wl1, bl, eodd, meven)
    return pl.pallas_call(
        body,
        grid=(EB // BB,),
        in_specs=[_pblk(128)] * 3 + [_wfull(w) for w in ws],
        out_specs=[_pblk(2048), _pblk(128)],
        out_shape=[jax.ShapeDtypeStruct((EB, 2048), jnp.float32),
                   jax.ShapeDtypeStruct((EB, 128), jnp.float32)],
    )(g_s, g_d, a_int, *ws)


def _edge_dn_first_tc(a_int, p):
    W = _bd(p["le_dn"]["W"])
    b = _bt(p["le_dn"]["b"])

    def body(a_ref, w, br, msg_ref):
        msg_ref[...] = jnp.maximum(a_ref[...] @ w[...] + br[...], 0.0)

    return pl.pallas_call(
        body,
        grid=(EB // BB,),
        in_specs=[_pblk(128), _wfull(W), _wfull(b)],
        out_specs=_pblk(2048),
        out_shape=jax.ShapeDtypeStruct((EB, 2048), jnp.float32),
    )(a_int, W, b)


def _edge_dn_later_tc(g_s, a_int, p):
    Wap = _bd(p["ln_ap"]["W"])
    bap = _bt(p["ln_ap"]["b"])
    Wld = _bd(p["le_dn"]["W"])
    bld = _bt(p["le_dn"]["b"])

    def body(s_ref, a_ref, w1, b1r, w2, b2r, msg_ref):
        msg_ref[...] = (
            jnp.maximum(s_ref[...] @ w1[...] + b1r[...], 0.0)
            + jnp.maximum(a_ref[...] @ w2[...] + b2r[...], 0.0))

    ws = (Wap, bap, Wld, bld)
    return pl.pallas_call(
        body,
        grid=(EB // BB,),
        in_specs=[_pblk(128)] * 2 + [_wfull(w) for w in ws],
        out_specs=_pblk(2048),
        out_shape=jax.ShapeDtypeStruct((EB, 2048), jnp.float32),
    )(g_s, a_int, *ws)


def _node_tc(x, aggr, cnt, pn, p1, p2):
    Wn = pn["W"]
    bn = pn["b"].reshape(1, -1)
    W1 = p1["W"]
    W1x, W1a = W1[0:2], W1[2:34]
    b1 = p1["b"].reshape(1, -1)
    W2 = p2["W"]
    b2 = p2["b"].reshape(1, -1)

    def body(x_ref, a_ref, c_ref, wn, bnr, w1x, w1a, b1r, w2, b2r, o_ref):
        x = x_ref[...]
        c = jnp.maximum(c_ref[...], 1.0)
        t = a_ref[...] / c + jnp.maximum(x @ wn[...] + bnr[...], 0.0)
        h = jnp.maximum(x @ w1x[...] + t @ w1a[...] + b1r[...], 0.0)
        pw = jax.nn.sigmoid(h @ w2[...] + b2r[...])
        o_ref[...] = jnp.concatenate([x[:, 0:1], pw], axis=1)

    ws = (Wn, bn, W1x, W1a, b1, W2, b2)
    nb = lambda w: pl.BlockSpec((BN, w), lambda i: (i, 0))
    return pl.pallas_call(
        body,
        grid=(N // BN,),
        in_specs=[nb(2), nb(D), nb(1)] + [_wfull(w) for w in ws],
        out_specs=nb(2),
        out_shape=jax.ShapeDtypeStruct((N, 2), jnp.float32),
    )(x, aggr, cnt, *ws)


def kernel(x_ue, x_ap, ei_up, ea_up, ei_dn, ea_dn, params):
    pad = E_PAD - E
    i32 = jnp.int32
    src_up = jnp.concatenate([ei_up[0].astype(i32), jnp.zeros((pad,), i32)])
    dst_up = jnp.concatenate([ei_up[1].astype(i32), jnp.full((pad,), N, i32)])
    src_dn = jnp.concatenate([ei_dn[0].astype(i32), jnp.zeros((pad,), i32)])
    dst_dn = jnp.concatenate([ei_dn[1].astype(i32), jnp.full((pad,), N, i32)])
    dst2d_up = dst_up.reshape(E_PAD // 128, 128)
    dst2d_dn = dst_dn.reshape(E_PAD // 128, 128)
    a_up = jnp.concatenate(
        [ea_up, jnp.zeros((pad, 2), jnp.float32)]).reshape(EB, 128)
    a_dn = jnp.concatenate(
        [ea_dn, jnp.zeros((pad, 2), jnp.float32)]).reshape(EB, 128)
    npad = N_PAD - N
    t_ue0 = jnp.concatenate([x_ue[:, 0], jnp.zeros((npad,), jnp.float32)])
    t_ue1 = jnp.concatenate([x_ue[:, 1], jnp.zeros((npad,), jnp.float32)])
    t_ap0 = jnp.concatenate([x_ap[:, 0], jnp.zeros((npad,), jnp.float32)])
    t_ap1 = jnp.concatenate([x_ap[:, 1], jnp.zeros((npad,), jnp.float32)])

    p0, p1 = params[0], params[1]

    gs, gd = _gather([t_ue0, t_ue1, src_up, t_ap0, t_ap1, dst_up], (1, 1))
    msg_up, a_up2 = _edge_up_tc(gs, gd, a_up, p0)
    aggr_up, cnt_up = _scatter_counts(msg_up, dst2d_up)
    x_ap1 = _node_tc(x_ap, aggr_up, cnt_up, p0["ln_ap"], p0["pm1"], p0["pm2"])
    msg_dn = _edge_dn_first_tc(a_dn, p0)
    aggr_dn, cnt_dn = _scatter_counts(msg_dn, dst2d_dn)
    x_ue1 = _node_tc(x_ue, aggr_dn, cnt_dn, p0["ln_ue"], p0["pm1"], p0["pm2"])

    t_ue1b = jnp.concatenate([x_ue1[:, 1], jnp.zeros((npad,), jnp.float32)])
    t_ap1b = jnp.concatenate([x_ap1[:, 1], jnp.zeros((npad,), jnp.float32)])
    gs2, gd2, gdn2 = _gather(
        [t_ue0, t_ue1b, src_up, t_ap0, t_ap1b, dst_up, src_dn], (1, 2))
    msg_up2, a_up3 = _edge_up_tc(gs2, gd2, a_up2, p1)
    aggr_up2 = _scatter(msg_up2, dst2d_up)
    x_ap2 = _node_tc(x_ap1, aggr_up2, cnt_up,
                     p1["ln_ap"], p1["pm1"], p1["pm2"])
    msg_dn2 = _edge_dn_later_tc(gdn2, a_dn, p1)
    aggr_dn2 = _scatter(msg_dn2, dst2d_dn)
    x_ue2 = _node_tc(x_ue1, aggr_dn2, cnt_dn,
                     p1["ln_ue"], p1["pm1"], p1["pm2"])

    ea_up_new = a_up3.reshape(E_PAD, 2)[:E]
    return (x_ue2, x_ap2, ea_up_new, ea_dn)
